# baseline (speedup 1.0000x reference)
import functools

import jax
import jax.numpy as jnp
from jax import lax
from jax.experimental import pallas as pl
from jax.experimental.pallas import tpu as pltpu
from jax.experimental.pallas import tpu_sc as plsc

_NC = 2
_NS = 16
_LANES = 16


def _firing_and_m(h, centers_i, log_sigma_i, W_m_i, b_m_i):
    inv2 = 0.5 * jnp.exp(-2.0 * log_sigma_i)
    p1 = inv2.T
    p2 = -(2.0 * centers_i * inv2).T
    p3 = jnp.sum(centers_i * centers_i * inv2, axis=-1)
    d = (
        jnp.dot(h * h, p1, preferred_element_type=jnp.float32,
                precision=lax.Precision.HIGHEST)
        + jnp.dot(h, p2, preferred_element_type=jnp.float32,
                  precision=lax.Precision.HIGHEST)
        + p3[None, :]
    )
    neg = -d
    mx = jnp.max(neg, axis=-1, keepdims=True)
    e = jnp.exp(neg - mx)
    firing = e / jnp.sum(e, axis=-1, keepdims=True)
    m = jnp.dot(h, W_m_i, preferred_element_type=jnp.float32) + b_m_i[None, :]
    return firing, m


def _tc_in_body(x_ref, W_in_ref, b_in_ref, c_ref, ls_ref, Wm_ref, bm_ref,
                h_ref, m_ref, f_ref):
    h = jnp.maximum(
        jnp.dot(x_ref[...], W_in_ref[...], preferred_element_type=jnp.float32)
        + b_in_ref[...][None, :],
        0.0,
    )
    firing, m = _firing_and_m(h, c_ref[...], ls_ref[...], Wm_ref[...], bm_ref[...])
    h_ref[...] = h
    m_ref[...] = m
    f_ref[...] = firing


def _tc_stats_body(a0m_ref, a1m_ref, a0t_ref, a1t_ref, hn_ref, sum_ref, sq_ref,
                   *, hid):
    norm = a0t_ref[...][:, 0] + a1t_ref[...][:, 0] + 1e-6
    h_new = (a0m_ref[...] + a1m_ref[...]) / norm[:, None]
    hn_ref[...] = h_new
    sum_ref[...] = jnp.sum(h_new, axis=0, keepdims=True)[None]
    sq_ref[...] = jnp.sum(h_new * h_new, axis=0, keepdims=True)[None]


def _bn_from_stats(h_prev, h_new, sums, sqs, gamma_i, beta_i, n):
    mean = jnp.sum(sums[...], axis=0) / n
    ex2 = jnp.sum(sqs[...], axis=0) / n
    var = ex2 - mean * mean
    hb = gamma_i[None, :] * (h_new - mean) / jnp.sqrt(var + 1e-5) + beta_i[None, :]
    return h_prev + jnp.maximum(hb, 0.0)


def _tc_mid_body(h_ref, hn_ref, sum_ref, sq_ref, g_ref, b_ref, c_ref, ls_ref,
                 Wm_ref, bm_ref, h_out_ref, m_ref, f_ref, *, n):
    h = _bn_from_stats(h_ref[...], hn_ref[...], sum_ref, sq_ref,
                       g_ref[...], b_ref[...], n)
    firing, m = _firing_and_m(h, c_ref[...], ls_ref[...], Wm_ref[...], bm_ref[...])
    h_out_ref[...] = h
    m_ref[...] = m
    f_ref[...] = firing


def _tc_out_body(h_ref, hn_ref, sum_ref, sq_ref, g_ref, b_ref, W1_ref, b1_ref,
                 W2_ref, b2_ref, out_ref, *, n):
    h = _bn_from_stats(h_ref[...], hn_ref[...], sum_ref, sq_ref,
                       g_ref[...], b_ref[...], n)
    hidden = jnp.maximum(
        jnp.dot(h, W1_ref[...], preferred_element_type=jnp.float32)
        + b1_ref[...][None, :],
        0.0,
    )
    logits = (
        jnp.dot(hidden, W2_ref[...], preferred_element_type=jnp.float32)
        + b2_ref[...][None, :]
    )
    mx = jnp.max(logits, axis=-1, keepdims=True)
    e = jnp.exp(logits - mx)
    out_ref[...] = e / jnp.sum(e, axis=-1, keepdims=True)


@functools.lru_cache(maxsize=None)
def _make_sc_edge(n, e, hid, r, chunk):
    tw = _LANES
    nw = _NC * _NS
    epw = e // nw
    n_chunks = epw // chunk
    z_stages_tot = n // chunk
    z_stages_per_tile = -(-z_stages_tot // _NS)
    mesh = plsc.VectorSubcoreMesh(core_axis_name="c", subcore_axis_name="s")

    assert n_chunks % 2 == 1 and n % chunk == 0 and n % _NS == 0
    half = (n_chunks - 1) // 2

    @functools.partial(
        pl.kernel,
        out_type=[
            jax.ShapeDtypeStruct((_NC, n, hid), jnp.float32),
            jax.ShapeDtypeStruct((_NC, n, tw), jnp.float32),
        ],
        mesh=mesh,
        compiler_params=pltpu.CompilerParams(
            use_tc_tiling_on_sc=False, needs_layout_passes=False
        ),
        scratch_types=[
            pltpu.VMEM((chunk,), jnp.int32),
            pltpu.VMEM((chunk,), jnp.int32),
            pltpu.VMEM((chunk,), jnp.int32),
            pltpu.VMEM((chunk,), jnp.int32),
            pltpu.VMEM((chunk,), jnp.int32),
            pltpu.VMEM((chunk,), jnp.int32),
            pltpu.VMEM((chunk, r), jnp.float32),
            pltpu.VMEM((chunk, r), jnp.float32),
            pltpu.VMEM((chunk, r), jnp.float32),
            pltpu.VMEM((chunk, r), jnp.float32),
            pltpu.VMEM((chunk, hid), jnp.float32),
            pltpu.VMEM((chunk, hid), jnp.float32),
            pltpu.VMEM((chunk, tw), jnp.float32),
            pltpu.VMEM((chunk, tw), jnp.float32),
            pltpu.VMEM_SHARED((n, hid), jnp.float32),
            pltpu.VMEM_SHARED((n, tw), jnp.float32),
            pltpu.SemaphoreType.DMA,
            pltpu.SemaphoreType.DMA,
            pltpu.SemaphoreType.DMA,
            pltpu.SemaphoreType.DMA,
            pltpu.SemaphoreType.DMA,
            pltpu.SemaphoreType.DMA,
        ],
    )
    def sc_edge(m_hbm, f_hbm, src3_hbm, dst3_hbm, out_m_hbm, out_t_hbm,
                sidx0, sidx1, didx0, didx1, sdix0, sdix1,
                fsrc0, fsrc1, fdst0, fdst1, outbuf0, outbuf1, tbuf0, tbuf1,
                acc_m, acc_t, isem0, isem1, gsem0, gsem1, ssem0, ssem1):
        cid = lax.axis_index("c")
        sid = lax.axis_index("s")
        wid = cid * _NS + sid
        sidx = (sidx0, sidx1)
        didx = (didx0, didx1)
        sdix = (sdix0, sdix1)
        fsrc = (fsrc0, fsrc1)
        fdst = (fdst0, fdst1)
        outbuf = (outbuf0, outbuf1)
        tbuf = (tbuf0, tbuf1)
        isem = (isem0, isem1)
        gsem = (gsem0, gsem1)
        ssem = (ssem0, ssem1)

        zvec = jnp.zeros((_LANES,), jnp.float32)

        for i in range(chunk):
            for j in range(hid // _LANES):
                outbuf0[i, pl.ds(j * _LANES, _LANES)] = zvec
            tbuf0[i, pl.ds(0, _LANES)] = zvec
            tbuf1[i, pl.ds(0, _LANES)] = zvec
        for k in range(z_stages_per_tile):
            st = k * _NS + sid

            @pl.when(st < z_stages_tot)
            def _zero_stage():
                pltpu.async_copy(outbuf0, acc_m.at[pl.ds(st * chunk, chunk)],
                                 gsem0)
                pltpu.async_copy(tbuf0, acc_t.at[pl.ds(st * chunk, chunk)],
                                 gsem0)
        for k in range(z_stages_per_tile):
            st = k * _NS + sid

            @pl.when(st < z_stages_tot)
            def _zero_drain():
                pltpu.make_async_copy(
                    outbuf0, acc_m.at[pl.ds(0, chunk)], gsem0).wait()
                pltpu.make_async_copy(
                    tbuf0, acc_t.at[pl.ds(0, chunk)], gsem0).wait()
        plsc.subcore_barrier()

        col0 = jnp.full((_LANES,), 0, jnp.int32)

        def fire_idx(k, p):
            pltpu.async_copy(src3_hbm.at[wid, k], sidx[p], isem[p])
            pltpu.async_copy(dst3_hbm.at[wid, k], didx[p], isem[p])

        def drain_idx(p):
            pltpu.make_async_copy(src3_hbm.at[0, 0], sidx[p], isem[p]).wait()
            pltpu.make_async_copy(src3_hbm.at[0, 0], didx[p], isem[p]).wait()

        def fire_gathers(p):
            pltpu.async_copy(m_hbm.at[sidx[p]], outbuf[p], gsem[p])
            pltpu.async_copy(f_hbm.at[sidx[p]], fsrc[p], gsem[p])
            pltpu.async_copy(f_hbm.at[didx[p]], fdst[p], gsem[p])

        def drain_gathers(p):
            pltpu.make_async_copy(m_hbm.at[pl.ds(0, chunk)], outbuf[p], gsem[p]).wait()
            pltpu.make_async_copy(f_hbm.at[pl.ds(0, chunk)], fsrc[p], gsem[p]).wait()
            pltpu.make_async_copy(f_hbm.at[pl.ds(0, chunk)], fdst[p], gsem[p]).wait()

        def drain_scatter(p):
            pltpu.make_async_copy(outbuf[p], acc_m.at[sdix[p]], ssem[p]).wait()
            pltpu.make_async_copy(tbuf[p], acc_t.at[sdix[p]], ssem[p]).wait()

        def copy_didx_to_sdix(p):
            for j in range(chunk // _LANES):
                sl = pl.ds(j * _LANES, _LANES)
                sdix[p][sl] = didx[p][sl]

        def compute_and_fire_scatter(p):
            def tgroup(g, carry2):
                e_vec = g * _LANES + lax.iota(jnp.int32, _LANES)
                acc_tv = jnp.zeros((_LANES,), jnp.float32)
                for rr in range(r):
                    rv = jnp.full((_LANES,), rr, jnp.int32)
                    a = plsc.load_gather(fsrc[p], [e_vec, rv])
                    b = plsc.load_gather(fdst[p], [e_vec, rv])
                    acc_tv = acc_tv + a * b
                plsc.store_scatter(tbuf[p], [e_vec, col0], acc_tv)
                for c16 in range(_LANES):
                    c = g * _LANES + c16
                    t = acc_tv[c16]
                    for j in range(hid // _LANES):
                        sl = pl.ds(j * _LANES, _LANES)
                        outbuf[p][c, sl] = outbuf[p][c, sl] * t
                return carry2

            lax.fori_loop(0, chunk // _LANES, tgroup, 0, unroll=True)
            pltpu.async_copy(outbuf[p], acc_m.at[sdix[p]], ssem[p], add=True)
            pltpu.async_copy(tbuf[p], acc_t.at[sdix[p]], ssem[p], add=True)

        def step(k, p, q, have_next, have_next2, have_prev_scatter):
            drain_gathers(p)
            copy_didx_to_sdix(p)
            if have_next2:
                @pl.when(k + 2 < n_chunks)
                def _prefetch_idx():
                    fire_idx(k + 2, p)
            if have_next:
                drain_idx(q)
                if have_prev_scatter:
                    drain_scatter(q)
                fire_gathers(q)
            compute_and_fire_scatter(p)

        fire_idx(0, 0)
        drain_idx(0)
        fire_gathers(0)
        fire_idx(1, 1)

        def do_pair(g, carry):
            a = 2 * g

            @pl.when(g == 0)
            def _first_pair():
                step(a, 0, 1, True, True, False)
                step(a + 1, 1, 0, True, True, True)

            @pl.when(g > 0)
            def _steady_pair():
                step(a, 0, 1, True, True, True)
                step(a + 1, 1, 0, True, True, True)
            return carry

        lax.fori_loop(0, half, do_pair, 0)
        step(n_chunks - 1, 0, 1, False, False, False)
        drain_scatter(1)
        drain_scatter(0)
        plsc.subcore_barrier()

        out_rows = n // _NS
        sl = pl.ds(sid * out_rows, out_rows)
        pltpu.async_copy(acc_m.at[sl], out_m_hbm.at[cid, sl], gsem0)
        pltpu.async_copy(acc_t.at[sl], out_t_hbm.at[cid, sl], gsem1)
        pltpu.make_async_copy(acc_m.at[sl], out_m_hbm.at[cid, sl], gsem0).wait()
        pltpu.make_async_copy(acc_t.at[sl], out_t_hbm.at[cid, sl], gsem1).wait()

    return sc_edge


def kernel(x, edge_index, W_in, b_in, centers, log_sigma, W_m, b_m, gamma,
           beta, W1, b1, W2, b2):
    n, in_c = x.shape
    e = edge_index.shape[1]
    hid = W_in.shape[1]
    num_l, r, _ = centers.shape
    out_c = W2.shape[1]
    chunk = 80
    br = 1000
    nb = n // br

    nw = _NC * _NS
    n_chunks = e // nw // chunk
    src3 = edge_index[0].reshape(nw, n_chunks, chunk)
    dst3 = edge_index[1].reshape(nw, n_chunks, chunk)

    f32 = jnp.float32

    def rows(w):
        return pl.BlockSpec((br, w), lambda i: (i, 0))

    def full2(a, b):
        return pl.BlockSpec((a, b), lambda i: (0, 0))

    def full1(a):
        return pl.BlockSpec((a,), lambda i: (0,))

    def full3(a, b, c):
        return pl.BlockSpec((a, b, c), lambda i: (0, 0, 0))

    tc_in = pl.pallas_call(
        _tc_in_body,
        grid=(nb,),
        in_specs=[rows(in_c), full2(in_c, hid), full1(hid), full2(r, hid),
                  full2(r, hid), full2(hid, hid), full1(hid)],
        out_specs=[rows(hid), rows(hid), rows(r)],
        out_shape=[
            jax.ShapeDtypeStruct((n, hid), f32),
            jax.ShapeDtypeStruct((n, hid), f32),
            jax.ShapeDtypeStruct((n, r), f32),
        ],
    )
    tc_stats = pl.pallas_call(
        functools.partial(_tc_stats_body, hid=hid),
        grid=(nb,),
        in_specs=[rows(hid), rows(hid), rows(_LANES), rows(_LANES)],
        out_specs=[rows(hid), pl.BlockSpec((1, 1, hid), lambda i: (i, 0, 0)),
                   pl.BlockSpec((1, 1, hid), lambda i: (i, 0, 0))],
        out_shape=[
            jax.ShapeDtypeStruct((n, hid), f32),
            jax.ShapeDtypeStruct((nb, 1, hid), f32),
            jax.ShapeDtypeStruct((nb, 1, hid), f32),
        ],
    )
    tc_mid = pl.pallas_call(
        functools.partial(_tc_mid_body, n=n),
        grid=(nb,),
        in_specs=[rows(hid), rows(hid), full3(nb, 1, hid), full3(nb, 1, hid),
                  full1(hid), full1(hid), full2(r, hid), full2(r, hid),
                  full2(hid, hid), full1(hid)],
        out_specs=[rows(hid), rows(hid), rows(r)],
        out_shape=[
            jax.ShapeDtypeStruct((n, hid), f32),
            jax.ShapeDtypeStruct((n, hid), f32),
            jax.ShapeDtypeStruct((n, r), f32),
        ],
    )
    tc_out = pl.pallas_call(
        functools.partial(_tc_out_body, n=n),
        grid=(nb,),
        in_specs=[rows(hid), rows(hid), full3(nb, 1, hid), full3(nb, 1, hid),
                  full1(hid), full1(hid), full2(hid, hid // 2), full1(hid // 2),
                  full2(hid // 2, out_c), full1(out_c)],
        out_specs=rows(out_c),
        out_shape=jax.ShapeDtypeStruct((n, out_c), f32),
    )
    sc_edge = _make_sc_edge(n, e, hid, r, chunk)

    h, m, firing = tc_in(x, W_in, b_in, centers[0], log_sigma[0], W_m[0], b_m[0])
    for i in range(num_l):
        agg_m, agg_t = sc_edge(m, firing, src3, dst3)
        hn, sums, sqs = tc_stats(agg_m[0], agg_m[1], agg_t[0], agg_t[1])
        if i + 1 < num_l:
            h, m, firing = tc_mid(
                h, hn, sums, sqs, gamma[i], beta[i],
                centers[i + 1], log_sigma[i + 1], W_m[i + 1], b_m[i + 1],
            )
        else:
            out = tc_out(h, hn, sums, sqs, gamma[i], beta[i], W1, b1, W2, b2)
    return out


# speedup vs baseline: 1.1144x; 1.1144x over previous
import functools

import jax
import jax.numpy as jnp
from jax import lax
from jax.experimental import pallas as pl
from jax.experimental.pallas import tpu as pltpu
from jax.experimental.pallas import tpu_sc as plsc

_NC = 2
_NS = 16
_LANES = 16


def _firing_and_m(h, centers_i, log_sigma_i, W_m_i, b_m_i):
    inv2 = 0.5 * jnp.exp(-2.0 * log_sigma_i)
    p1 = inv2.T
    p2 = -(2.0 * centers_i * inv2).T
    p3 = jnp.sum(centers_i * centers_i * inv2, axis=-1)
    d = (
        jnp.dot(h * h, p1, preferred_element_type=jnp.float32,
                precision=lax.Precision.HIGHEST)
        + jnp.dot(h, p2, preferred_element_type=jnp.float32,
                  precision=lax.Precision.HIGHEST)
        + p3[None, :]
    )
    neg = -d
    mx = jnp.max(neg, axis=-1, keepdims=True)
    e = jnp.exp(neg - mx)
    firing = e / jnp.sum(e, axis=-1, keepdims=True)
    m = jnp.dot(h, W_m_i, preferred_element_type=jnp.float32) + b_m_i[None, :]
    return firing, m


def _tc_in_body(x_ref, W_in_ref, b_in_ref, c_ref, ls_ref, Wm_ref, bm_ref,
                h_ref, m_ref, f_ref):
    h = jnp.maximum(
        jnp.dot(x_ref[...], W_in_ref[...], preferred_element_type=jnp.float32)
        + b_in_ref[...][None, :],
        0.0,
    )
    firing, m = _firing_and_m(h, c_ref[...], ls_ref[...], Wm_ref[...], bm_ref[...])
    h_ref[...] = h
    m_ref[...] = jnp.concatenate([m, firing], axis=1)
    f_ref[...] = firing


def _tc_stats_body(a0_ref, a1_ref, hn_ref, sum_ref, sq_ref, *, hid):
    s = a0_ref[...] + a1_ref[...]
    h_new = s[:, :hid] / (s[:, hid] + 1e-6)[:, None]
    hn_ref[...] = h_new
    sum_ref[...] = jnp.sum(h_new, axis=0, keepdims=True)[None]
    sq_ref[...] = jnp.sum(h_new * h_new, axis=0, keepdims=True)[None]


def _bn_from_stats(h_prev, h_new, sums, sqs, gamma_i, beta_i, n):
    mean = jnp.sum(sums[...], axis=0) / n
    ex2 = jnp.sum(sqs[...], axis=0) / n
    var = ex2 - mean * mean
    hb = gamma_i[None, :] * (h_new - mean) / jnp.sqrt(var + 1e-5) + beta_i[None, :]
    return h_prev + jnp.maximum(hb, 0.0)


def _tc_mid_body(h_ref, hn_ref, sum_ref, sq_ref, g_ref, b_ref, c_ref, ls_ref,
                 Wm_ref, bm_ref, h_out_ref, m_ref, f_ref, *, n):
    h = _bn_from_stats(h_ref[...], hn_ref[...], sum_ref, sq_ref,
                       g_ref[...], b_ref[...], n)
    firing, m = _firing_and_m(h, c_ref[...], ls_ref[...], Wm_ref[...], bm_ref[...])
    h_out_ref[...] = h
    m_ref[...] = jnp.concatenate([m, firing], axis=1)
    f_ref[...] = firing


def _tc_out_body(h_ref, hn_ref, sum_ref, sq_ref, g_ref, b_ref, W1_ref, b1_ref,
                 W2_ref, b2_ref, out_ref, *, n):
    h = _bn_from_stats(h_ref[...], hn_ref[...], sum_ref, sq_ref,
                       g_ref[...], b_ref[...], n)
    hidden = jnp.maximum(
        jnp.dot(h, W1_ref[...], preferred_element_type=jnp.float32)
        + b1_ref[...][None, :],
        0.0,
    )
    logits = (
        jnp.dot(hidden, W2_ref[...], preferred_element_type=jnp.float32)
        + b2_ref[...][None, :]
    )
    mx = jnp.max(logits, axis=-1, keepdims=True)
    e = jnp.exp(logits - mx)
    out_ref[...] = e / jnp.sum(e, axis=-1, keepdims=True)


@functools.lru_cache(maxsize=None)
def _make_sc_edge(n, e, hid, r, chunk):
    aw = hid + r
    nw = _NC * _NS
    epw = e // nw
    n_chunks = epw // chunk
    z_stages_tot = n // chunk
    z_stages_per_tile = -(-z_stages_tot // _NS)
    mesh = plsc.VectorSubcoreMesh(core_axis_name="c", subcore_axis_name="s")

    assert n_chunks % 2 == 1 and n % chunk == 0 and n % _NS == 0
    half = (n_chunks - 1) // 2

    @functools.partial(
        pl.kernel,
        out_type=jax.ShapeDtypeStruct((_NC, n, aw), jnp.float32),
        mesh=mesh,
        compiler_params=pltpu.CompilerParams(
            use_tc_tiling_on_sc=False, needs_layout_passes=False
        ),
        scratch_types=[
            pltpu.VMEM((chunk,), jnp.int32),
            pltpu.VMEM((chunk,), jnp.int32),
            pltpu.VMEM((chunk,), jnp.int32),
            pltpu.VMEM((chunk,), jnp.int32),
            pltpu.VMEM((chunk,), jnp.int32),
            pltpu.VMEM((chunk,), jnp.int32),
            pltpu.VMEM((chunk, r), jnp.float32),
            pltpu.VMEM((chunk, r), jnp.float32),
            pltpu.VMEM((chunk, aw), jnp.float32),
            pltpu.VMEM((chunk, aw), jnp.float32),
            pltpu.VMEM_SHARED((n, aw), jnp.float32),
            pltpu.SemaphoreType.DMA,
            pltpu.SemaphoreType.DMA,
            pltpu.SemaphoreType.DMA,
            pltpu.SemaphoreType.DMA,
            pltpu.SemaphoreType.DMA,
            pltpu.SemaphoreType.DMA,
        ],
    )
    def sc_edge(msg_hbm, f_hbm, src3_hbm, dst3_hbm, out_hbm,
                sidx0, sidx1, didx0, didx1, sdix0, sdix1,
                fdst0, fdst1, outbuf0, outbuf1,
                acc, isem0, isem1, gsem0, gsem1, ssem0, ssem1):
        cid = lax.axis_index("c")
        sid = lax.axis_index("s")
        wid = cid * _NS + sid
        sidx = (sidx0, sidx1)
        didx = (didx0, didx1)
        sdix = (sdix0, sdix1)
        fdst = (fdst0, fdst1)
        outbuf = (outbuf0, outbuf1)
        isem = (isem0, isem1)
        gsem = (gsem0, gsem1)
        ssem = (ssem0, ssem1)

        zvec = jnp.zeros((_LANES,), jnp.float32)

        for i in range(chunk):
            for j in range(aw // _LANES):
                outbuf0[i, pl.ds(j * _LANES, _LANES)] = zvec
        for k in range(z_stages_per_tile):
            st = k * _NS + sid

            @pl.when(st < z_stages_tot)
            def _zero_stage():
                pltpu.async_copy(outbuf0, acc.at[pl.ds(st * chunk, chunk)],
                                 gsem0)
        for k in range(z_stages_per_tile):
            st = k * _NS + sid

            @pl.when(st < z_stages_tot)
            def _zero_drain():
                pltpu.make_async_copy(
                    outbuf0, acc.at[pl.ds(0, chunk)], gsem0).wait()
        plsc.subcore_barrier()

        col_t = jnp.full((_LANES,), hid, jnp.int32)

        def fire_idx(k, p):
            pltpu.async_copy(src3_hbm.at[wid, k], sidx[p], isem[p])
            pltpu.async_copy(dst3_hbm.at[wid, k], didx[p], isem[p])

        def drain_idx(p):
            pltpu.make_async_copy(src3_hbm.at[0, 0], sidx[p], isem[p]).wait()
            pltpu.make_async_copy(src3_hbm.at[0, 0], didx[p], isem[p]).wait()

        def fire_gathers(p):
            pltpu.async_copy(msg_hbm.at[sidx[p]], outbuf[p], gsem[p])
            pltpu.async_copy(f_hbm.at[didx[p]], fdst[p], gsem[p])

        def drain_gathers(p):
            pltpu.make_async_copy(msg_hbm.at[pl.ds(0, chunk)], outbuf[p],
                                  gsem[p]).wait()
            pltpu.make_async_copy(f_hbm.at[pl.ds(0, chunk)], fdst[p],
                                  gsem[p]).wait()

        def drain_scatter(p):
            pltpu.make_async_copy(outbuf[p], acc.at[sdix[p]], ssem[p]).wait()

        def copy_didx_to_sdix(p):
            for j in range(chunk // _LANES):
                sl = pl.ds(j * _LANES, _LANES)
                sdix[p][sl] = didx[p][sl]

        def compute_and_fire_scatter(p):
            def tgroup(g, carry2):
                e_vec = g * _LANES + lax.iota(jnp.int32, _LANES)
                acc_tv = jnp.zeros((_LANES,), jnp.float32)
                for rr in range(r):
                    rv = jnp.full((_LANES,), rr, jnp.int32)
                    a = plsc.load_gather(outbuf[p], [e_vec, rv + hid])
                    b = plsc.load_gather(fdst[p], [e_vec, rv])
                    acc_tv = acc_tv + a * b
                plsc.store_scatter(outbuf[p], [e_vec, col_t], acc_tv)
                for c16 in range(_LANES):
                    c = g * _LANES + c16
                    t = acc_tv[c16]
                    for j in range(hid // _LANES):
                        sl = pl.ds(j * _LANES, _LANES)
                        outbuf[p][c, sl] = outbuf[p][c, sl] * t
                return carry2

            lax.fori_loop(0, chunk // _LANES, tgroup, 0)
            pltpu.async_copy(outbuf[p], acc.at[sdix[p]], ssem[p], add=True)

        def step(k, p, q, have_next, have_next2, have_prev_scatter):
            drain_gathers(p)
            copy_didx_to_sdix(p)
            if have_next2:
                @pl.when(k + 2 < n_chunks)
                def _prefetch_idx():
                    fire_idx(k + 2, p)
            if have_next:
                drain_idx(q)
                if have_prev_scatter:
                    drain_scatter(q)
                fire_gathers(q)
            compute_and_fire_scatter(p)

        fire_idx(0, 0)
        drain_idx(0)
        fire_gathers(0)
        fire_idx(1, 1)

        def do_pair(g, carry):
            a = 2 * g

            @pl.when(g == 0)
            def _first_pair():
                step(a, 0, 1, True, True, False)
                step(a + 1, 1, 0, True, True, True)

            @pl.when(g > 0)
            def _steady_pair():
                step(a, 0, 1, True, True, True)
                step(a + 1, 1, 0, True, True, True)
            return carry

        lax.fori_loop(0, half, do_pair, 0)
        step(n_chunks - 1, 0, 1, False, False, False)
        drain_scatter(1)
        drain_scatter(0)
        plsc.subcore_barrier()

        out_rows = n // _NS
        sl = pl.ds(sid * out_rows, out_rows)
        pltpu.async_copy(acc.at[sl], out_hbm.at[cid, sl], gsem0)
        pltpu.make_async_copy(acc.at[sl], out_hbm.at[cid, sl], gsem0).wait()

    return sc_edge


def kernel(x, edge_index, W_in, b_in, centers, log_sigma, W_m, b_m, gamma,
           beta, W1, b1, W2, b2):
    n, in_c = x.shape
    e = edge_index.shape[1]
    hid = W_in.shape[1]
    num_l, r, _ = centers.shape
    out_c = W2.shape[1]
    aw = hid + r
    chunk = 80
    br = 1000
    nb = n // br

    nw = _NC * _NS
    n_chunks = e // nw // chunk
    src3 = edge_index[0].reshape(nw, n_chunks, chunk)
    dst3 = edge_index[1].reshape(nw, n_chunks, chunk)

    f32 = jnp.float32

    def rows(w):
        return pl.BlockSpec((br, w), lambda i: (i, 0))

    def full2(a, b):
        return pl.BlockSpec((a, b), lambda i: (0, 0))

    def full1(a):
        return pl.BlockSpec((a,), lambda i: (0,))

    def full3(a, b, c):
        return pl.BlockSpec((a, b, c), lambda i: (0, 0, 0))

    tc_in = pl.pallas_call(
        _tc_in_body,
        grid=(nb,),
        in_specs=[rows(in_c), full2(in_c, hid), full1(hid), full2(r, hid),
                  full2(r, hid), full2(hid, hid), full1(hid)],
        out_specs=[rows(hid), rows(aw), rows(r)],
        out_shape=[
            jax.ShapeDtypeStruct((n, hid), f32),
            jax.ShapeDtypeStruct((n, aw), f32),
            jax.ShapeDtypeStruct((n, r), f32),
        ],
    )
    tc_stats = pl.pallas_call(
        functools.partial(_tc_stats_body, hid=hid),
        grid=(nb,),
        in_specs=[rows(aw), rows(aw)],
        out_specs=[rows(hid), pl.BlockSpec((1, 1, hid), lambda i: (i, 0, 0)),
                   pl.BlockSpec((1, 1, hid), lambda i: (i, 0, 0))],
        out_shape=[
            jax.ShapeDtypeStruct((n, hid), f32),
            jax.ShapeDtypeStruct((nb, 1, hid), f32),
            jax.ShapeDtypeStruct((nb, 1, hid), f32),
        ],
    )
    tc_mid = pl.pallas_call(
        functools.partial(_tc_mid_body, n=n),
        grid=(nb,),
        in_specs=[rows(hid), rows(hid), full3(nb, 1, hid), full3(nb, 1, hid),
                  full1(hid), full1(hid), full2(r, hid), full2(r, hid),
                  full2(hid, hid), full1(hid)],
        out_specs=[rows(hid), rows(aw), rows(r)],
        out_shape=[
            jax.ShapeDtypeStruct((n, hid), f32),
            jax.ShapeDtypeStruct((n, aw), f32),
            jax.ShapeDtypeStruct((n, r), f32),
        ],
    )
    tc_out = pl.pallas_call(
        functools.partial(_tc_out_body, n=n),
        grid=(nb,),
        in_specs=[rows(hid), rows(hid), full3(nb, 1, hid), full3(nb, 1, hid),
                  full1(hid), full1(hid), full2(hid, hid // 2), full1(hid // 2),
                  full2(hid // 2, out_c), full1(out_c)],
        out_specs=rows(out_c),
        out_shape=jax.ShapeDtypeStruct((n, out_c), f32),
    )
    sc_edge = _make_sc_edge(n, e, hid, r, chunk)

    h, m, firing = tc_in(x, W_in, b_in, centers[0], log_sigma[0], W_m[0], b_m[0])
    for i in range(num_l):
        agg = sc_edge(m, firing, src3, dst3)
        hn, sums, sqs = tc_stats(agg[0], agg[1])
        if i + 1 < num_l:
            h, m, firing = tc_mid(
                h, hn, sums, sqs, gamma[i], beta[i],
                centers[i + 1], log_sigma[i + 1], W_m[i + 1], b_m[i + 1],
            )
        else:
            out = tc_out(h, hn, sums, sqs, gamma[i], beta[i], W1, b1, W2, b2)
    return out


# speedup vs baseline: 1.2132x; 1.0887x over previous
import functools

import jax
import jax.numpy as jnp
from jax import lax
from jax.experimental import pallas as pl
from jax.experimental.pallas import tpu as pltpu
from jax.experimental.pallas import tpu_sc as plsc

_NC = 2
_NS = 16
_LANES = 16


def _firing_and_m(h, centers_i, log_sigma_i, W_m_i, b_m_i):
    inv2 = 0.5 * jnp.exp(-2.0 * log_sigma_i)
    p1 = inv2.T
    p2 = -(2.0 * centers_i * inv2).T
    p3 = jnp.sum(centers_i * centers_i * inv2, axis=-1)
    d = (
        jnp.dot(h * h, p1, preferred_element_type=jnp.float32,
                precision=lax.Precision.HIGHEST)
        + jnp.dot(h, p2, preferred_element_type=jnp.float32,
                  precision=lax.Precision.HIGHEST)
        + p3[None, :]
    )
    neg = -d
    mx = jnp.max(neg, axis=-1, keepdims=True)
    e = jnp.exp(neg - mx)
    firing = e / jnp.sum(e, axis=-1, keepdims=True)
    m = jnp.dot(h, W_m_i, preferred_element_type=jnp.float32) + b_m_i[None, :]
    return firing, m


def _tc_in_body(x_ref, W_in_ref, b_in_ref, c_ref, ls_ref, Wm_ref, bm_ref,
                h_ref, m_ref, f_ref):
    h = jnp.maximum(
        jnp.dot(x_ref[...], W_in_ref[...], preferred_element_type=jnp.float32)
        + b_in_ref[...][None, :],
        0.0,
    )
    firing, m = _firing_and_m(h, c_ref[...], ls_ref[...], Wm_ref[...], bm_ref[...])
    h_ref[...] = h
    m_ref[...] = m
    f_ref[...] = firing


def _tc_stats_body(a0m_ref, a1m_ref, a0t_ref, a1t_ref, hn_ref, sum_ref, sq_ref,
                   *, hid):
    norm = a0t_ref[...][:, 0] + a1t_ref[...][:, 0] + 1e-6
    h_new = (a0m_ref[...] + a1m_ref[...]) / norm[:, None]
    hn_ref[...] = h_new
    sum_ref[...] = jnp.sum(h_new, axis=0, keepdims=True)[None]
    sq_ref[...] = jnp.sum(h_new * h_new, axis=0, keepdims=True)[None]


def _bn_from_stats(h_prev, h_new, sums, sqs, gamma_i, beta_i, n):
    mean = jnp.sum(sums[...], axis=0) / n
    ex2 = jnp.sum(sqs[...], axis=0) / n
    var = ex2 - mean * mean
    hb = gamma_i[None, :] * (h_new - mean) / jnp.sqrt(var + 1e-5) + beta_i[None, :]
    return h_prev + jnp.maximum(hb, 0.0)


def _tc_mid_body(h_ref, hn_ref, sum_ref, sq_ref, g_ref, b_ref, c_ref, ls_ref,
                 Wm_ref, bm_ref, h_out_ref, m_ref, f_ref, *, n):
    h = _bn_from_stats(h_ref[...], hn_ref[...], sum_ref, sq_ref,
                       g_ref[...], b_ref[...], n)
    firing, m = _firing_and_m(h, c_ref[...], ls_ref[...], Wm_ref[...], bm_ref[...])
    h_out_ref[...] = h
    m_ref[...] = m
    f_ref[...] = firing


def _tc_out_body(h_ref, hn_ref, sum_ref, sq_ref, g_ref, b_ref, W1_ref, b1_ref,
                 W2_ref, b2_ref, out_ref, *, n):
    h = _bn_from_stats(h_ref[...], hn_ref[...], sum_ref, sq_ref,
                       g_ref[...], b_ref[...], n)
    hidden = jnp.maximum(
        jnp.dot(h, W1_ref[...], preferred_element_type=jnp.float32)
        + b1_ref[...][None, :],
        0.0,
    )
    logits = (
        jnp.dot(hidden, W2_ref[...], preferred_element_type=jnp.float32)
        + b2_ref[...][None, :]
    )
    mx = jnp.max(logits, axis=-1, keepdims=True)
    e = jnp.exp(logits - mx)
    out_ref[...] = e / jnp.sum(e, axis=-1, keepdims=True)


@functools.lru_cache(maxsize=None)
def _make_sc_edge(n, e, hid, r, chunk):
    tw = _LANES
    nw = _NC * _NS
    epw = e // nw
    n_chunks = epw // chunk
    z_stages_tot = n // chunk
    z_stages_per_tile = -(-z_stages_tot // _NS)
    mesh = plsc.VectorSubcoreMesh(core_axis_name="c", subcore_axis_name="s")

    assert n_chunks % 2 == 1 and n % chunk == 0 and n % _NS == 0
    half = (n_chunks - 1) // 2

    @functools.partial(
        pl.kernel,
        out_type=[
            jax.ShapeDtypeStruct((_NC, n, hid), jnp.float32),
            jax.ShapeDtypeStruct((_NC, n, tw), jnp.float32),
        ],
        mesh=mesh,
        compiler_params=pltpu.CompilerParams(
            use_tc_tiling_on_sc=False, needs_layout_passes=False
        ),
        scratch_types=[
            pltpu.VMEM((chunk,), jnp.int32),
            pltpu.VMEM((chunk,), jnp.int32),
            pltpu.VMEM((chunk,), jnp.int32),
            pltpu.VMEM((chunk,), jnp.int32),
            pltpu.VMEM((chunk,), jnp.int32),
            pltpu.VMEM((chunk,), jnp.int32),
            pltpu.VMEM((chunk, r), jnp.float32),
            pltpu.VMEM((chunk, r), jnp.float32),
            pltpu.VMEM((chunk, r), jnp.float32),
            pltpu.VMEM((chunk, r), jnp.float32),
            pltpu.VMEM((chunk, hid), jnp.float32),
            pltpu.VMEM((chunk, hid), jnp.float32),
            pltpu.VMEM((chunk, tw), jnp.float32),
            pltpu.VMEM((chunk, tw), jnp.float32),
            pltpu.VMEM_SHARED((n, hid), jnp.float32),
            pltpu.VMEM_SHARED((n, tw), jnp.float32),
            pltpu.SemaphoreType.DMA,
            pltpu.SemaphoreType.DMA,
            pltpu.SemaphoreType.DMA,
            pltpu.SemaphoreType.DMA,
            pltpu.SemaphoreType.DMA,
            pltpu.SemaphoreType.DMA,
        ],
    )
    def sc_edge(m_hbm, f_hbm, src3_hbm, dst3_hbm, out_m_hbm, out_t_hbm,
                sidx0, sidx1, didx0, didx1, sdix0, sdix1,
                fsrc0, fsrc1, fdst0, fdst1, outbuf0, outbuf1, tbuf0, tbuf1,
                acc_m, acc_t, isem0, isem1, gsem0, gsem1, ssem0, ssem1):
        cid = lax.axis_index("c")
        sid = lax.axis_index("s")
        wid = cid * _NS + sid
        sidx = (sidx0, sidx1)
        didx = (didx0, didx1)
        sdix = (sdix0, sdix1)
        fsrc = (fsrc0, fsrc1)
        fdst = (fdst0, fdst1)
        outbuf = (outbuf0, outbuf1)
        tbuf = (tbuf0, tbuf1)
        isem = (isem0, isem1)
        gsem = (gsem0, gsem1)
        ssem = (ssem0, ssem1)

        zvec = jnp.zeros((_LANES,), jnp.float32)

        for i in range(chunk):
            for j in range(hid // _LANES):
                outbuf0[i, pl.ds(j * _LANES, _LANES)] = zvec
            tbuf0[i, pl.ds(0, _LANES)] = zvec
            tbuf1[i, pl.ds(0, _LANES)] = zvec
        for k in range(z_stages_per_tile):
            st = k * _NS + sid

            @pl.when(st < z_stages_tot)
            def _zero_stage():
                pltpu.async_copy(outbuf0, acc_m.at[pl.ds(st * chunk, chunk)],
                                 gsem0)
                pltpu.async_copy(tbuf0, acc_t.at[pl.ds(st * chunk, chunk)],
                                 gsem0)
        for k in range(z_stages_per_tile):
            st = k * _NS + sid

            @pl.when(st < z_stages_tot)
            def _zero_drain():
                pltpu.make_async_copy(
                    outbuf0, acc_m.at[pl.ds(0, chunk)], gsem0).wait()
                pltpu.make_async_copy(
                    tbuf0, acc_t.at[pl.ds(0, chunk)], gsem0).wait()
        plsc.subcore_barrier()

        col0 = jnp.full((_LANES,), 0, jnp.int32)

        def fire_idx(k, p):
            pltpu.async_copy(src3_hbm.at[wid, k], sidx[p], isem[p])
            pltpu.async_copy(dst3_hbm.at[wid, k], didx[p], isem[p])

        def drain_idx(p):
            pltpu.make_async_copy(src3_hbm.at[0, 0], sidx[p], isem[p]).wait()
            pltpu.make_async_copy(src3_hbm.at[0, 0], didx[p], isem[p]).wait()

        def fire_gathers(p):
            pltpu.async_copy(m_hbm.at[sidx[p]], outbuf[p], gsem[p])
            pltpu.async_copy(f_hbm.at[sidx[p]], fsrc[p], gsem[p])
            pltpu.async_copy(f_hbm.at[didx[p]], fdst[p], gsem[p])

        def drain_gathers(p):
            pltpu.make_async_copy(m_hbm.at[pl.ds(0, chunk)], outbuf[p], gsem[p]).wait()
            pltpu.make_async_copy(f_hbm.at[pl.ds(0, chunk)], fsrc[p], gsem[p]).wait()
            pltpu.make_async_copy(f_hbm.at[pl.ds(0, chunk)], fdst[p], gsem[p]).wait()

        def drain_scatter(p):
            pltpu.make_async_copy(outbuf[p], acc_m.at[sdix[p]], ssem[p]).wait()
            pltpu.make_async_copy(tbuf[p], acc_t.at[sdix[p]], ssem[p]).wait()

        def copy_didx_to_sdix(p):
            for j in range(chunk // _LANES):
                sl = pl.ds(j * _LANES, _LANES)
                sdix[p][sl] = didx[p][sl]

        def compute_and_fire_scatter(p):
            @plsc.parallel_loop(0, chunk // _LANES, unroll=2)
            def tgroup(g):
                e_vec = g * _LANES + lax.iota(jnp.int32, _LANES)
                acc_tv = jnp.zeros((_LANES,), jnp.float32)
                for rr in range(r):
                    rv = jnp.full((_LANES,), rr, jnp.int32)
                    a = plsc.load_gather(fsrc[p], [e_vec, rv])
                    b = plsc.load_gather(fdst[p], [e_vec, rv])
                    acc_tv = acc_tv + a * b
                plsc.store_scatter(tbuf[p], [e_vec, col0], acc_tv)
                for c16 in range(_LANES):
                    c = g * _LANES + c16
                    t = acc_tv[c16]
                    for j in range(hid // _LANES):
                        sl = pl.ds(j * _LANES, _LANES)
                        outbuf[p][c, sl] = outbuf[p][c, sl] * t
            pltpu.async_copy(outbuf[p], acc_m.at[sdix[p]], ssem[p], add=True)
            pltpu.async_copy(tbuf[p], acc_t.at[sdix[p]], ssem[p], add=True)

        def step(k, p, q, have_next, have_next2, have_prev_scatter):
            drain_gathers(p)
            copy_didx_to_sdix(p)
            if have_next2:
                @pl.when(k + 2 < n_chunks)
                def _prefetch_idx():
                    fire_idx(k + 2, p)
            if have_next:
                drain_idx(q)
                if have_prev_scatter:
                    drain_scatter(q)
                fire_gathers(q)
            compute_and_fire_scatter(p)

        fire_idx(0, 0)
        drain_idx(0)
        fire_gathers(0)
        fire_idx(1, 1)

        def do_pair(g, carry):
            a = 2 * g

            @pl.when(g == 0)
            def _first_pair():
                step(a, 0, 1, True, True, False)
                step(a + 1, 1, 0, True, True, True)

            @pl.when(g > 0)
            def _steady_pair():
                step(a, 0, 1, True, True, True)
                step(a + 1, 1, 0, True, True, True)
            return carry

        lax.fori_loop(0, half, do_pair, 0)
        step(n_chunks - 1, 0, 1, False, False, False)
        drain_scatter(1)
        drain_scatter(0)
        plsc.subcore_barrier()

        out_rows = n // _NS
        sl = pl.ds(sid * out_rows, out_rows)
        pltpu.async_copy(acc_m.at[sl], out_m_hbm.at[cid, sl], gsem0)
        pltpu.async_copy(acc_t.at[sl], out_t_hbm.at[cid, sl], gsem1)
        pltpu.make_async_copy(acc_m.at[sl], out_m_hbm.at[cid, sl], gsem0).wait()
        pltpu.make_async_copy(acc_t.at[sl], out_t_hbm.at[cid, sl], gsem1).wait()

    return sc_edge


def kernel(x, edge_index, W_in, b_in, centers, log_sigma, W_m, b_m, gamma,
           beta, W1, b1, W2, b2):
    n, in_c = x.shape
    e = edge_index.shape[1]
    hid = W_in.shape[1]
    num_l, r, _ = centers.shape
    out_c = W2.shape[1]
    chunk = 80
    br = 1000
    nb = n // br

    nw = _NC * _NS
    n_chunks = e // nw // chunk
    src3 = edge_index[0].reshape(nw, n_chunks, chunk)
    dst3 = edge_index[1].reshape(nw, n_chunks, chunk)

    f32 = jnp.float32

    def rows(w):
        return pl.BlockSpec((br, w), lambda i: (i, 0))

    def full2(a, b):
        return pl.BlockSpec((a, b), lambda i: (0, 0))

    def full1(a):
        return pl.BlockSpec((a,), lambda i: (0,))

    def full3(a, b, c):
        return pl.BlockSpec((a, b, c), lambda i: (0, 0, 0))

    tc_in = pl.pallas_call(
        _tc_in_body,
        grid=(nb,),
        in_specs=[rows(in_c), full2(in_c, hid), full1(hid), full2(r, hid),
                  full2(r, hid), full2(hid, hid), full1(hid)],
        out_specs=[rows(hid), rows(hid), rows(r)],
        out_shape=[
            jax.ShapeDtypeStruct((n, hid), f32),
            jax.ShapeDtypeStruct((n, hid), f32),
            jax.ShapeDtypeStruct((n, r), f32),
        ],
    )
    tc_stats = pl.pallas_call(
        functools.partial(_tc_stats_body, hid=hid),
        grid=(nb,),
        in_specs=[rows(hid), rows(hid), rows(_LANES), rows(_LANES)],
        out_specs=[rows(hid), pl.BlockSpec((1, 1, hid), lambda i: (i, 0, 0)),
                   pl.BlockSpec((1, 1, hid), lambda i: (i, 0, 0))],
        out_shape=[
            jax.ShapeDtypeStruct((n, hid), f32),
            jax.ShapeDtypeStruct((nb, 1, hid), f32),
            jax.ShapeDtypeStruct((nb, 1, hid), f32),
        ],
    )
    tc_mid = pl.pallas_call(
        functools.partial(_tc_mid_body, n=n),
        grid=(nb,),
        in_specs=[rows(hid), rows(hid), full3(nb, 1, hid), full3(nb, 1, hid),
                  full1(hid), full1(hid), full2(r, hid), full2(r, hid),
                  full2(hid, hid), full1(hid)],
        out_specs=[rows(hid), rows(hid), rows(r)],
        out_shape=[
            jax.ShapeDtypeStruct((n, hid), f32),
            jax.ShapeDtypeStruct((n, hid), f32),
            jax.ShapeDtypeStruct((n, r), f32),
        ],
    )
    tc_out = pl.pallas_call(
        functools.partial(_tc_out_body, n=n),
        grid=(nb,),
        in_specs=[rows(hid), rows(hid), full3(nb, 1, hid), full3(nb, 1, hid),
                  full1(hid), full1(hid), full2(hid, hid // 2), full1(hid // 2),
                  full2(hid // 2, out_c), full1(out_c)],
        out_specs=rows(out_c),
        out_shape=jax.ShapeDtypeStruct((n, out_c), f32),
    )
    sc_edge = _make_sc_edge(n, e, hid, r, chunk)

    h, m, firing = tc_in(x, W_in, b_in, centers[0], log_sigma[0], W_m[0], b_m[0])
    for i in range(num_l):
        agg_m, agg_t = sc_edge(m, firing, src3, dst3)
        hn, sums, sqs = tc_stats(agg_m[0], agg_m[1], agg_t[0], agg_t[1])
        if i + 1 < num_l:
            h, m, firing = tc_mid(
                h, hn, sums, sqs, gamma[i], beta[i],
                centers[i + 1], log_sigma[i + 1], W_m[i + 1], b_m[i + 1],
            )
        else:
            out = tc_out(h, hn, sums, sqs, gamma[i], beta[i], W1, b1, W2, b2)
    return out


# speedup vs baseline: 1.2632x; 1.0412x over previous
import functools

import jax
import jax.numpy as jnp
from jax import lax
from jax.experimental import pallas as pl
from jax.experimental.pallas import tpu as pltpu
from jax.experimental.pallas import tpu_sc as plsc

_NC = 2
_NS = 16
_LANES = 16


def _firing_and_m(h, centers_i, log_sigma_i, W_m_i, b_m_i):
    inv2 = 0.5 * jnp.exp(-2.0 * log_sigma_i)
    p1 = inv2.T
    p2 = -(2.0 * centers_i * inv2).T
    p3 = jnp.sum(centers_i * centers_i * inv2, axis=-1)
    d = (
        jnp.dot(h * h, p1, preferred_element_type=jnp.float32,
                precision=lax.Precision.HIGHEST)
        + jnp.dot(h, p2, preferred_element_type=jnp.float32,
                  precision=lax.Precision.HIGHEST)
        + p3[None, :]
    )
    neg = -d
    mx = jnp.max(neg, axis=-1, keepdims=True)
    e = jnp.exp(neg - mx)
    firing = e / jnp.sum(e, axis=-1, keepdims=True)
    m = jnp.dot(h, W_m_i, preferred_element_type=jnp.float32) + b_m_i[None, :]
    return firing, m


def _tc_in_body(x_ref, W_in_ref, b_in_ref, c_ref, ls_ref, Wm_ref, bm_ref,
                h_ref, m_ref, f_ref):
    h = jnp.maximum(
        jnp.dot(x_ref[...], W_in_ref[...], preferred_element_type=jnp.float32)
        + b_in_ref[...][None, :],
        0.0,
    )
    firing, m = _firing_and_m(h, c_ref[...], ls_ref[...], Wm_ref[...], bm_ref[...])
    h_ref[...] = h
    m_ref[...] = m
    f_ref[...] = firing


def _tc_stats_body(a0m_ref, a1m_ref, a0t_ref, a1t_ref, hn_ref, sum_ref, sq_ref,
                   *, hid):
    norm = a0t_ref[...][:, 0] + a1t_ref[...][:, 0] + 1e-6
    h_new = (a0m_ref[...] + a1m_ref[...]) / norm[:, None]
    hn_ref[...] = h_new
    sum_ref[...] = jnp.sum(h_new, axis=0, keepdims=True)[None]
    sq_ref[...] = jnp.sum(h_new * h_new, axis=0, keepdims=True)[None]


def _bn_from_stats(h_prev, h_new, sums, sqs, gamma_i, beta_i, n):
    mean = jnp.sum(sums[...], axis=0) / n
    ex2 = jnp.sum(sqs[...], axis=0) / n
    var = ex2 - mean * mean
    hb = gamma_i[None, :] * (h_new - mean) / jnp.sqrt(var + 1e-5) + beta_i[None, :]
    return h_prev + jnp.maximum(hb, 0.0)


def _tc_mid_body(h_ref, hn_ref, sum_ref, sq_ref, g_ref, b_ref, c_ref, ls_ref,
                 Wm_ref, bm_ref, h_out_ref, m_ref, f_ref, *, n):
    h = _bn_from_stats(h_ref[...], hn_ref[...], sum_ref, sq_ref,
                       g_ref[...], b_ref[...], n)
    firing, m = _firing_and_m(h, c_ref[...], ls_ref[...], Wm_ref[...], bm_ref[...])
    h_out_ref[...] = h
    m_ref[...] = m
    f_ref[...] = firing


def _tc_out_body(h_ref, hn_ref, sum_ref, sq_ref, g_ref, b_ref, W1_ref, b1_ref,
                 W2_ref, b2_ref, out_ref, *, n):
    h = _bn_from_stats(h_ref[...], hn_ref[...], sum_ref, sq_ref,
                       g_ref[...], b_ref[...], n)
    hidden = jnp.maximum(
        jnp.dot(h, W1_ref[...], preferred_element_type=jnp.float32)
        + b1_ref[...][None, :],
        0.0,
    )
    logits = (
        jnp.dot(hidden, W2_ref[...], preferred_element_type=jnp.float32)
        + b2_ref[...][None, :]
    )
    mx = jnp.max(logits, axis=-1, keepdims=True)
    e = jnp.exp(logits - mx)
    out_ref[...] = e / jnp.sum(e, axis=-1, keepdims=True)


@functools.lru_cache(maxsize=None)
def _make_sc_edge(n, e, hid, r, chunk):
    tw = _LANES
    nw = _NC * _NS
    epw = e // nw
    n_chunks = epw // chunk
    z_stages_tot = n // chunk
    z_stages_per_tile = -(-z_stages_tot // _NS)
    mesh = plsc.VectorSubcoreMesh(core_axis_name="c", subcore_axis_name="s")

    assert n_chunks % 2 == 1 and n % chunk == 0 and n % _NS == 0
    half = (n_chunks - 1) // 2

    @functools.partial(
        pl.kernel,
        out_type=[
            jax.ShapeDtypeStruct((_NC, n, hid), jnp.float32),
            jax.ShapeDtypeStruct((_NC, n, tw), jnp.float32),
        ],
        mesh=mesh,
        compiler_params=pltpu.CompilerParams(
            use_tc_tiling_on_sc=False, needs_layout_passes=False
        ),
        scratch_types=[
            pltpu.VMEM((chunk,), jnp.int32),
            pltpu.VMEM((chunk,), jnp.int32),
            pltpu.VMEM((chunk,), jnp.int32),
            pltpu.VMEM((chunk,), jnp.int32),
            pltpu.VMEM((chunk,), jnp.int32),
            pltpu.VMEM((chunk,), jnp.int32),
            pltpu.VMEM((chunk, r), jnp.float32),
            pltpu.VMEM((chunk, r), jnp.float32),
            pltpu.VMEM((chunk, r), jnp.float32),
            pltpu.VMEM((chunk, r), jnp.float32),
            pltpu.VMEM((chunk, hid), jnp.float32),
            pltpu.VMEM((chunk, hid), jnp.float32),
            pltpu.VMEM((chunk, tw), jnp.float32),
            pltpu.VMEM((chunk, tw), jnp.float32),
            pltpu.VMEM_SHARED((n, hid), jnp.float32),
            pltpu.VMEM_SHARED((n, tw), jnp.float32),
            pltpu.SemaphoreType.DMA,
            pltpu.SemaphoreType.DMA,
            pltpu.SemaphoreType.DMA,
            pltpu.SemaphoreType.DMA,
            pltpu.SemaphoreType.DMA,
            pltpu.SemaphoreType.DMA,
        ],
    )
    def sc_edge(m_hbm, f_hbm, src3_hbm, dst3_hbm, out_m_hbm, out_t_hbm,
                sidx0, sidx1, didx0, didx1, sdix0, sdix1,
                fsrc0, fsrc1, fdst0, fdst1, outbuf0, outbuf1, tbuf0, tbuf1,
                acc_m, acc_t, isem0, isem1, gsem0, gsem1, ssem0, ssem1):
        cid = lax.axis_index("c")
        sid = lax.axis_index("s")
        wid = cid * _NS + sid
        sidx = (sidx0, sidx1)
        didx = (didx0, didx1)
        sdix = (sdix0, sdix1)
        fsrc = (fsrc0, fsrc1)
        fdst = (fdst0, fdst1)
        outbuf = (outbuf0, outbuf1)
        tbuf = (tbuf0, tbuf1)
        isem = (isem0, isem1)
        gsem = (gsem0, gsem1)
        ssem = (ssem0, ssem1)

        zvec = jnp.zeros((_LANES,), jnp.float32)

        for i in range(chunk):
            for j in range(hid // _LANES):
                outbuf0[i, pl.ds(j * _LANES, _LANES)] = zvec
            tbuf0[i, pl.ds(0, _LANES)] = zvec
            tbuf1[i, pl.ds(0, _LANES)] = zvec
        for k in range(z_stages_per_tile):
            st = k * _NS + sid

            @pl.when(st < z_stages_tot)
            def _zero_stage():
                pltpu.async_copy(outbuf0, acc_m.at[pl.ds(st * chunk, chunk)],
                                 gsem0)
                pltpu.async_copy(tbuf0, acc_t.at[pl.ds(st * chunk, chunk)],
                                 gsem0)
        for k in range(z_stages_per_tile):
            st = k * _NS + sid

            @pl.when(st < z_stages_tot)
            def _zero_drain():
                pltpu.make_async_copy(
                    outbuf0, acc_m.at[pl.ds(0, chunk)], gsem0).wait()
                pltpu.make_async_copy(
                    tbuf0, acc_t.at[pl.ds(0, chunk)], gsem0).wait()
        plsc.subcore_barrier()

        col0 = jnp.full((_LANES,), 0, jnp.int32)

        def fire_idx(k, p):
            pltpu.async_copy(src3_hbm.at[wid, k], sidx[p], isem[p])
            pltpu.async_copy(dst3_hbm.at[wid, k], didx[p], isem[p])

        def drain_idx(p):
            pltpu.make_async_copy(src3_hbm.at[0, 0], sidx[p], isem[p]).wait()
            pltpu.make_async_copy(src3_hbm.at[0, 0], didx[p], isem[p]).wait()

        def fire_gathers(p):
            pltpu.async_copy(m_hbm.at[sidx[p]], outbuf[p], gsem[p])
            pltpu.async_copy(f_hbm.at[sidx[p]], fsrc[p], gsem[p])
            pltpu.async_copy(f_hbm.at[didx[p]], fdst[p], gsem[p])

        def drain_gathers(p):
            pltpu.make_async_copy(m_hbm.at[pl.ds(0, chunk)], outbuf[p], gsem[p]).wait()
            pltpu.make_async_copy(f_hbm.at[pl.ds(0, chunk)], fsrc[p], gsem[p]).wait()
            pltpu.make_async_copy(f_hbm.at[pl.ds(0, chunk)], fdst[p], gsem[p]).wait()

        def drain_scatter(p):
            pltpu.make_async_copy(outbuf[p], acc_m.at[sdix[p]], ssem[p]).wait()
            pltpu.make_async_copy(tbuf[p], acc_t.at[sdix[p]], ssem[p]).wait()

        def copy_didx_to_sdix(p):
            for j in range(chunk // _LANES):
                sl = pl.ds(j * _LANES, _LANES)
                sdix[p][sl] = didx[p][sl]

        def compute_and_fire_scatter(p):
            @plsc.parallel_loop(0, chunk // _LANES, unroll=2)
            def tgroup(g):
                e_vec = g * _LANES + lax.iota(jnp.int32, _LANES)
                acc_tv = jnp.zeros((_LANES,), jnp.float32)
                for rr in range(r):
                    rv = jnp.full((_LANES,), rr, jnp.int32)
                    a = plsc.load_gather(fsrc[p], [e_vec, rv])
                    b = plsc.load_gather(fdst[p], [e_vec, rv])
                    acc_tv = acc_tv + a * b
                plsc.store_scatter(tbuf[p], [e_vec, col0], acc_tv)
                for c16 in range(_LANES):
                    c = g * _LANES + c16
                    t = acc_tv[c16]
                    for j in range(hid // _LANES):
                        sl = pl.ds(j * _LANES, _LANES)
                        outbuf[p][c, sl] = outbuf[p][c, sl] * t
            pltpu.async_copy(outbuf[p], acc_m.at[sdix[p]], ssem[p], add=True)
            pltpu.async_copy(tbuf[p], acc_t.at[sdix[p]], ssem[p], add=True)

        def step(k, p, q, have_next, have_next2, have_prev_scatter):
            drain_gathers(p)
            copy_didx_to_sdix(p)
            if have_next2:
                @pl.when(k + 2 < n_chunks)
                def _prefetch_idx():
                    fire_idx(k + 2, p)
            if have_next:
                drain_idx(q)
                if have_prev_scatter:
                    drain_scatter(q)
                fire_gathers(q)
            compute_and_fire_scatter(p)

        fire_idx(0, 0)
        drain_idx(0)
        fire_gathers(0)
        fire_idx(1, 1)

        def do_pair(g, carry):
            a = 2 * g

            @pl.when(g == 0)
            def _first_pair():
                step(a, 0, 1, True, True, False)
                step(a + 1, 1, 0, True, True, True)

            @pl.when(g > 0)
            def _steady_pair():
                step(a, 0, 1, True, True, True)
                step(a + 1, 1, 0, True, True, True)
            return carry

        lax.fori_loop(0, half, do_pair, 0)
        step(n_chunks - 1, 0, 1, False, False, False)
        drain_scatter(1)
        drain_scatter(0)
        plsc.subcore_barrier()

        out_rows = n // _NS
        sl = pl.ds(sid * out_rows, out_rows)
        pltpu.async_copy(acc_m.at[sl], out_m_hbm.at[cid, sl], gsem0)
        pltpu.async_copy(acc_t.at[sl], out_t_hbm.at[cid, sl], gsem1)
        pltpu.make_async_copy(acc_m.at[sl], out_m_hbm.at[cid, sl], gsem0).wait()
        pltpu.make_async_copy(acc_t.at[sl], out_t_hbm.at[cid, sl], gsem1).wait()

    return sc_edge


def kernel(x, edge_index, W_in, b_in, centers, log_sigma, W_m, b_m, gamma,
           beta, W1, b1, W2, b2):
    n, in_c = x.shape
    e = edge_index.shape[1]
    hid = W_in.shape[1]
    num_l, r, _ = centers.shape
    out_c = W2.shape[1]
    chunk = 80
    br = 2000
    nb = n // br

    nw = _NC * _NS
    n_chunks = e // nw // chunk
    src3 = edge_index[0].reshape(nw, n_chunks, chunk)
    dst3 = edge_index[1].reshape(nw, n_chunks, chunk)

    f32 = jnp.float32

    def rows(w):
        return pl.BlockSpec((br, w), lambda i: (i, 0))

    def full2(a, b):
        return pl.BlockSpec((a, b), lambda i: (0, 0))

    def full1(a):
        return pl.BlockSpec((a,), lambda i: (0,))

    def full3(a, b, c):
        return pl.BlockSpec((a, b, c), lambda i: (0, 0, 0))

    tc_in = pl.pallas_call(
        _tc_in_body,
        grid=(nb,),
        in_specs=[rows(in_c), full2(in_c, hid), full1(hid), full2(r, hid),
                  full2(r, hid), full2(hid, hid), full1(hid)],
        out_specs=[rows(hid), rows(hid), rows(r)],
        out_shape=[
            jax.ShapeDtypeStruct((n, hid), f32),
            jax.ShapeDtypeStruct((n, hid), f32),
            jax.ShapeDtypeStruct((n, r), f32),
        ],
    )
    tc_stats = pl.pallas_call(
        functools.partial(_tc_stats_body, hid=hid),
        grid=(nb,),
        in_specs=[rows(hid), rows(hid), rows(_LANES), rows(_LANES)],
        out_specs=[rows(hid), pl.BlockSpec((1, 1, hid), lambda i: (i, 0, 0)),
                   pl.BlockSpec((1, 1, hid), lambda i: (i, 0, 0))],
        out_shape=[
            jax.ShapeDtypeStruct((n, hid), f32),
            jax.ShapeDtypeStruct((nb, 1, hid), f32),
            jax.ShapeDtypeStruct((nb, 1, hid), f32),
        ],
    )
    tc_mid = pl.pallas_call(
        functools.partial(_tc_mid_body, n=n),
        grid=(nb,),
        in_specs=[rows(hid), rows(hid), full3(nb, 1, hid), full3(nb, 1, hid),
                  full1(hid), full1(hid), full2(r, hid), full2(r, hid),
                  full2(hid, hid), full1(hid)],
        out_specs=[rows(hid), rows(hid), rows(r)],
        out_shape=[
            jax.ShapeDtypeStruct((n, hid), f32),
            jax.ShapeDtypeStruct((n, hid), f32),
            jax.ShapeDtypeStruct((n, r), f32),
        ],
    )
    tc_out = pl.pallas_call(
        functools.partial(_tc_out_body, n=n),
        grid=(nb,),
        in_specs=[rows(hid), rows(hid), full3(nb, 1, hid), full3(nb, 1, hid),
                  full1(hid), full1(hid), full2(hid, hid // 2), full1(hid // 2),
                  full2(hid // 2, out_c), full1(out_c)],
        out_specs=rows(out_c),
        out_shape=jax.ShapeDtypeStruct((n, out_c), f32),
    )
    sc_edge = _make_sc_edge(n, e, hid, r, chunk)

    h, m, firing = tc_in(x, W_in, b_in, centers[0], log_sigma[0], W_m[0], b_m[0])
    for i in range(num_l):
        agg_m, agg_t = sc_edge(m, firing, src3, dst3)
        hn, sums, sqs = tc_stats(agg_m[0], agg_m[1], agg_t[0], agg_t[1])
        if i + 1 < num_l:
            h, m, firing = tc_mid(
                h, hn, sums, sqs, gamma[i], beta[i],
                centers[i + 1], log_sigma[i + 1], W_m[i + 1], b_m[i + 1],
            )
        else:
            out = tc_out(h, hn, sums, sqs, gamma[i], beta[i], W1, b1, W2, b2)
    return out
